# baseline (speedup 1.0000x reference)
import functools

import numpy as np

import jax
import jax.numpy as jnp
from jax import lax
from jax.experimental import pallas as pl
from jax.experimental.pallas import tpu as pltpu
from jax.experimental.pallas import tpu_sc as plsc

N = 10000
E = 320000
D = 128
DW = D // 2
DE = 16

K = 40
NCH = E // K
NTILES = 16
NW = 2 * NTILES
CH_PER_TILE = NCH // NW
ROWS_MOST = 624
ROWS_LAST = N - 15 * ROWS_MOST
ZCH = 16
IB = 25
NIB = CH_PER_TILE // IB
PAIRS = CH_PER_TILE // 2

MASK_HI = np.int32(-65536)

_q = []
for _g in range(D // 32):
    _q += [16 * _g + _k for _k in range(16)]
    _q += [64 + 16 * _g + _k for _k in range(16)]
QPERM = np.array(_q, dtype=np.int32)


def _lin_body(x_ref, w_ref, b_ref, o_ref):
    o_ref[...] = (
        jnp.dot(x_ref[...], w_ref[...], preferred_element_type=jnp.float32)
        + b_ref[...]
    )


def _linear(x, w, b, block_rows):
    rows, din = x.shape
    dout = w.shape[1]
    grid = rows // block_rows
    return pl.pallas_call(
        _lin_body,
        grid=(grid,),
        in_specs=[
            pl.BlockSpec((block_rows, din), lambda i: (i, 0)),
            pl.BlockSpec((din, dout), lambda i: (0, 0)),
            pl.BlockSpec((1, dout), lambda i: (0, 0)),
        ],
        out_specs=pl.BlockSpec((block_rows, dout), lambda i: (i, 0)),
        out_shape=jax.ShapeDtypeStruct((rows, dout), jnp.float32),
    )(x, w, b.reshape(1, dout))


def _eenc_body(x_ref, w_ref, b_ref, o_ref):
    o = (
        jnp.dot(x_ref[...], w_ref[...], preferred_element_type=jnp.float32)
        + b_ref[...]
    )
    lo = jax.lax.bitcast_convert_type(
        o[:, :DW].astype(jnp.bfloat16), jnp.uint16
    ).astype(jnp.int32)
    hi = jax.lax.bitcast_convert_type(
        o[:, DW:].astype(jnp.bfloat16), jnp.uint16
    ).astype(jnp.int32)
    o_ref[...] = (hi << 16) | lo


def _edge_encoder(ea, w, b):
    br = 4000
    return pl.pallas_call(
        _eenc_body,
        grid=(E // br,),
        in_specs=[
            pl.BlockSpec((br, DE), lambda i: (i, 0)),
            pl.BlockSpec((DE, D), lambda i: (0, 0)),
            pl.BlockSpec((1, D), lambda i: (0, 0)),
        ],
        out_specs=pl.BlockSpec((br, DW), lambda i: (i, 0)),
        out_shape=jax.ShapeDtypeStruct((E, DW), jnp.int32),
    )(ea, w, b.reshape(1, D))


def _mlp_body(h_ref, a_ref, w1_ref, b1_ref, w2_ref, b2_ref, o_ref, *, final_relu):
    h = h_ref[...] + a_ref[0] + a_ref[1]
    t = jnp.maximum(
        jnp.dot(h, w1_ref[...], preferred_element_type=jnp.float32) + b1_ref[...],
        0.0,
    )
    o = jnp.dot(t, w2_ref[...], preferred_element_type=jnp.float32) + b2_ref[...]
    if final_relu:
        o = jnp.maximum(o, 0.0)
    o_ref[...] = o


def _gine_mlp(h, agg2, w1, b1, w2, b2, final_relu):
    block_rows = 2000
    grid = N // block_rows
    return pl.pallas_call(
        functools.partial(_mlp_body, final_relu=final_relu),
        grid=(grid,),
        in_specs=[
            pl.BlockSpec((block_rows, D), lambda i: (i, 0)),
            pl.BlockSpec((2, block_rows, D), lambda i: (0, i, 0)),
            pl.BlockSpec((D, D), lambda i: (0, 0)),
            pl.BlockSpec((1, D), lambda i: (0, 0)),
            pl.BlockSpec((D, D), lambda i: (0, 0)),
            pl.BlockSpec((1, D), lambda i: (0, 0)),
        ],
        out_specs=pl.BlockSpec((block_rows, D), lambda i: (i, 0)),
        out_shape=jax.ShapeDtypeStruct((N, D), jnp.float32),
    )(h, agg2, w1, b1.reshape(1, D), w2, b2.reshape(1, D))


def _sc_aggregate(xe, ee, src2d, dst2d):
    mesh = plsc.VectorSubcoreMesh(core_axis_name="c", subcore_axis_name="s")

    @functools.partial(
        pl.kernel,
        out_type=jax.ShapeDtypeStruct((2, N, D), jnp.float32),
        mesh=mesh,
        scratch_types=[
            pltpu.VMEM((2, IB, K), jnp.int32),
            pltpu.VMEM((2, IB, K), jnp.int32),
            pltpu.VMEM((K, D), jnp.float32),
            pltpu.VMEM((K, D), jnp.float32),
            pltpu.VMEM((K, DW), jnp.int32),
            pltpu.VMEM((K, DW), jnp.int32),
            pltpu.VMEM((K, D), jnp.float32),
            pltpu.VMEM((K, D), jnp.float32),
            pltpu.VMEM_SHARED((N, D), jnp.float32),
            pltpu.SemaphoreType.DMA,
            pltpu.SemaphoreType.DMA,
            pltpu.SemaphoreType.DMA,
            pltpu.SemaphoreType.DMA,
            pltpu.SemaphoreType.DMA,
            pltpu.SemaphoreType.DMA,
        ],
    )
    def body(xe_hbm, ee_hbm, src_hbm, dst_hbm, out_hbm,
             src_v, dst_v, gx0_v, gx1_v, ge0_v, ge1_v, ms0_v, ms1_v,
             acc_sh, sgx0, sgx1, sge0, sge1, ssc0, ssc1):
        gx = (gx0_v, gx1_v)
        ge = (ge0_v, ge1_v)
        ms = (ms0_v, ms1_v)
        sgx = (sgx0, sgx1)
        sge = (sge0, sge1)
        ssc = (ssc0, ssc1)
        cid = lax.axis_index("c")
        sid = lax.axis_index("s")
        wid = cid * NTILES + sid

        row0 = sid * ROWS_MOST
        n_zch = jnp.where(sid == NTILES - 1, ROWS_LAST // ZCH, ROWS_MOST // ZCH)

        def zrow(r, carry):
            for c in range(D // 16):
                ms0_v[r, pl.ds(c * 16, 16)] = jnp.zeros((16,), jnp.float32)
            return carry

        lax.fori_loop(0, ZCH, zrow, 0)

        def zcp(i, carry):
            pltpu.sync_copy(
                ms0_v.at[pl.ds(0, ZCH)],
                acc_sh.at[pl.ds(row0 + i * ZCH, ZCH)],
            )
            return carry

        lax.fori_loop(0, n_zch, zcp, 0)
        plsc.subcore_barrier()

        ee_base = wid * CH_PER_TILE

        def idx_row(ref, j):
            return ref.at[(j // IB) % 2, j % IB]

        def wait_i32(sem, buf):
            pltpu.make_async_copy(ee_hbm.at[pl.ds(0, K)], buf, sem).wait()

        def wait_f32(sem, buf):
            pltpu.make_async_copy(xe_hbm.at[pl.ds(0, K)], buf, sem).wait()

        pltpu.sync_copy(src_hbm.at[wid * NIB], src_v.at[0])
        pltpu.sync_copy(dst_hbm.at[wid * NIB], dst_v.at[0])
        for b in range(2):
            pltpu.async_copy(xe_hbm.at[src_v.at[0, b]], gx[b], sgx[b])
            pltpu.async_copy(ee_hbm.at[pl.ds((ee_base + b) * K, K)], ge[b], sge[b])

        def pair(j2, carry):
            for b in range(2):
                j = 2 * j2 + b
                @pl.when(j >= 2)
                def _wait_sc():
                    wait_f32(ssc[b], ms[b])

                wait_f32(sgx[b], gx[b])
                wait_i32(sge[b], ge[b])

                @pl.when(jnp.logical_and(j % IB == 3, j < (NIB - 1) * IB))
                def _refill():
                    nblk = j // IB + 1
                    pltpu.sync_copy(src_hbm.at[wid * NIB + nblk],
                                    src_v.at[nblk % 2])
                    pltpu.sync_copy(dst_hbm.at[wid * NIB + nblk],
                                    dst_v.at[nblk % 2])

                @plsc.parallel_loop(0, K, unroll=4)
                def _row(r):
                    for g in range(D // 32):
                        ew = ge[b][r, pl.ds(g * 16, 16)]
                        elo = jax.lax.bitcast_convert_type(ew << 16, jnp.float32)
                        ehi = jax.lax.bitcast_convert_type(ew & MASK_HI,
                                                           jnp.float32)
                        slo = pl.ds(32 * g, 16)
                        shi = pl.ds(32 * g + 16, 16)
                        ms[b][r, slo] = jnp.maximum(gx[b][r, slo] + elo, 0.0)
                        ms[b][r, shi] = jnp.maximum(gx[b][r, shi] + ehi, 0.0)
                pltpu.async_copy(ms[b], acc_sh.at[idx_row(dst_v, j)],
                                 ssc[b], add=True)

                @pl.when(j + 2 < CH_PER_TILE)
                def _issue_next():
                    jn = j + 2
                    pltpu.async_copy(xe_hbm.at[idx_row(src_v, jn)],
                                     gx[b], sgx[b])
                    pltpu.async_copy(ee_hbm.at[pl.ds((ee_base + jn) * K, K)],
                                     ge[b], sge[b])
            return carry

        lax.fori_loop(0, PAIRS, pair, 0)
        for b in range(2):
            wait_f32(ssc[b], ms[b])
        plsc.subcore_barrier()

        def ocp(i, carry):
            r0 = row0 + i * ZCH
            pltpu.sync_copy(acc_sh.at[pl.ds(r0, ZCH)], ms0_v.at[pl.ds(0, ZCH)])
            pltpu.sync_copy(ms0_v.at[pl.ds(0, ZCH)], out_hbm.at[cid, pl.ds(r0, ZCH)])
            return carry

        lax.fori_loop(0, n_zch, ocp, 0)

    return body(xe, ee, src2d, dst2d)


def kernel(x, edge_index, edge_attr, Wn, bn, We, be,
           W1a, b1a, W2a, b2a, W1b, b1b, W2b, b2b):
    src2d = edge_index[0].astype(jnp.int32).reshape(NW * NIB, IB, K)
    dst2d = edge_index[1].astype(jnp.int32).reshape(NW * NIB, IB, K)

    q = QPERM
    xe = _linear(x, Wn[:, q], bn[q], block_rows=2000)
    ee = _edge_encoder(edge_attr, We, be)

    parts1 = _sc_aggregate(xe, ee, src2d, dst2d)
    h = _gine_mlp(xe, parts1, W1a[q, :], b1a, W2a[:, q], b2a[q],
                  final_relu=True)

    parts2 = _sc_aggregate(h, ee, src2d, dst2d)
    out = _gine_mlp(h, parts2, W1b[q, :], b1b, W2b, b2b, final_relu=False)
    return out


# speedup vs baseline: 1.0731x; 1.0731x over previous
import functools

import jax
import jax.numpy as jnp
from jax import lax
from jax.experimental import pallas as pl
from jax.experimental.pallas import tpu as pltpu
from jax.experimental.pallas import tpu_sc as plsc

N = 10000
E = 320000
D = 128
DE = 16

K = 40
NCH = E // K
NTILES = 16
NW = 2 * NTILES
CH_PER_TILE = NCH // NW
ROWS_MOST = 624
ROWS_LAST = N - 15 * ROWS_MOST
ZCH = 16
IB = 25
NIB = CH_PER_TILE // IB
PAIRS = CH_PER_TILE // 2


def _lin_body(x_ref, w_ref, b_ref, o_ref):
    o_ref[...] = (
        jnp.dot(x_ref[...], w_ref[...], preferred_element_type=jnp.float32)
        + b_ref[...]
    )


def _linear(x, w, b, block_rows):
    rows, din = x.shape
    dout = w.shape[1]
    grid = rows // block_rows
    return pl.pallas_call(
        _lin_body,
        grid=(grid,),
        in_specs=[
            pl.BlockSpec((block_rows, din), lambda i: (i, 0)),
            pl.BlockSpec((din, dout), lambda i: (0, 0)),
            pl.BlockSpec((1, dout), lambda i: (0, 0)),
        ],
        out_specs=pl.BlockSpec((block_rows, dout), lambda i: (i, 0)),
        out_shape=jax.ShapeDtypeStruct((rows, dout), jnp.float32),
    )(x, w, b.reshape(1, dout))


def _mlp_body(h_ref, a_ref, w1_ref, b1_ref, w2_ref, b2_ref, o_ref, *, final_relu):
    h = h_ref[...] + a_ref[0] + a_ref[1]
    t = jnp.maximum(
        jnp.dot(h, w1_ref[...], preferred_element_type=jnp.float32) + b1_ref[...],
        0.0,
    )
    o = jnp.dot(t, w2_ref[...], preferred_element_type=jnp.float32) + b2_ref[...]
    if final_relu:
        o = jnp.maximum(o, 0.0)
    o_ref[...] = o


def _gine_mlp(h, agg2, w1, b1, w2, b2, final_relu):
    block_rows = 2000
    grid = N // block_rows
    return pl.pallas_call(
        functools.partial(_mlp_body, final_relu=final_relu),
        grid=(grid,),
        in_specs=[
            pl.BlockSpec((block_rows, D), lambda i: (i, 0)),
            pl.BlockSpec((2, block_rows, D), lambda i: (0, i, 0)),
            pl.BlockSpec((D, D), lambda i: (0, 0)),
            pl.BlockSpec((1, D), lambda i: (0, 0)),
            pl.BlockSpec((D, D), lambda i: (0, 0)),
            pl.BlockSpec((1, D), lambda i: (0, 0)),
        ],
        out_specs=pl.BlockSpec((block_rows, D), lambda i: (i, 0)),
        out_shape=jax.ShapeDtypeStruct((N, D), jnp.float32),
    )(h, agg2, w1, b1.reshape(1, D), w2, b2.reshape(1, D))


def _sc_aggregate(xe, ee, src2d, dst2d):
    mesh = plsc.VectorSubcoreMesh(core_axis_name="c", subcore_axis_name="s")

    @functools.partial(
        pl.kernel,
        out_type=jax.ShapeDtypeStruct((2, N, D), jnp.float32),
        mesh=mesh,
        scratch_types=[
            pltpu.VMEM((2, IB, K), jnp.int32),
            pltpu.VMEM((2, IB, K), jnp.int32),
            pltpu.VMEM((K, D), jnp.float32),
            pltpu.VMEM((K, D), jnp.float32),
            pltpu.VMEM((K, D), jnp.float32),
            pltpu.VMEM((K, D), jnp.float32),
            pltpu.VMEM((K, D), jnp.float32),
            pltpu.VMEM((K, D), jnp.float32),
            pltpu.VMEM_SHARED((N, D), jnp.float32),
            pltpu.SemaphoreType.DMA,
            pltpu.SemaphoreType.DMA,
            pltpu.SemaphoreType.DMA,
            pltpu.SemaphoreType.DMA,
            pltpu.SemaphoreType.DMA,
            pltpu.SemaphoreType.DMA,
            pltpu.SemaphoreType.DMA,
        ],
    )
    def body(xe_hbm, ee_hbm, src_hbm, dst_hbm, out_hbm,
             src_v, dst_v, gx0_v, gx1_v, ge0_v, ge1_v, ms0_v, ms1_v,
             acc_sh, sgx0, sgx1, sge0, sge1, ssc0, ssc1, szr):
        gx = (gx0_v, gx1_v)
        ge = (ge0_v, ge1_v)
        ms = (ms0_v, ms1_v)
        sgx = (sgx0, sgx1)
        sge = (sge0, sge1)
        ssc = (ssc0, ssc1)
        cid = lax.axis_index("c")
        sid = lax.axis_index("s")
        wid = cid * NTILES + sid

        row0 = sid * ROWS_MOST
        n_zch = jnp.where(sid == NTILES - 1, ROWS_LAST // ZCH, ROWS_MOST // ZCH)

        def zrow(r, carry):
            for c in range(D // 16):
                ms0_v[r, pl.ds(c * 16, 16)] = jnp.zeros((16,), jnp.float32)
            return carry

        lax.fori_loop(0, ZCH, zrow, 0)

        def zcp(i, carry):
            pltpu.async_copy(
                ms0_v.at[pl.ds(0, ZCH)],
                acc_sh.at[pl.ds(row0 + i * ZCH, ZCH)],
                szr,
            )
            return carry

        lax.fori_loop(0, n_zch, zcp, 0)
        pltpu.make_async_copy(
            xe_hbm.at[pl.ds(0, ROWS_MOST)],
            acc_sh.at[pl.ds(row0, ROWS_MOST)],
            szr,
        ).wait()

        @pl.when(sid == NTILES - 1)
        def _drain_last():
            pltpu.make_async_copy(
                xe_hbm.at[pl.ds(0, ROWS_LAST - ROWS_MOST)],
                acc_sh.at[pl.ds(row0, ROWS_LAST - ROWS_MOST)],
                szr,
            ).wait()

        plsc.subcore_barrier()

        ee_base = wid * CH_PER_TILE

        def idx_row(ref, j):
            return ref.at[(j // IB) % 2, j % IB]

        def wait_sem(sem, buf):
            pltpu.make_async_copy(ee_hbm.at[pl.ds(0, K)], buf, sem).wait()

        pltpu.sync_copy(src_hbm.at[wid * NIB], src_v.at[0])
        pltpu.sync_copy(dst_hbm.at[wid * NIB], dst_v.at[0])
        for b in range(2):
            pltpu.async_copy(xe_hbm.at[src_v.at[0, b]], gx[b], sgx[b])
            pltpu.async_copy(ee_hbm.at[pl.ds((ee_base + b) * K, K)], ge[b], sge[b])

        def pair(j2, carry):
            for b in range(2):
                j = 2 * j2 + b
                @pl.when(j >= 2)
                def _wait_sc():
                    wait_sem(ssc[b], ms[b])

                wait_sem(sgx[b], gx[b])
                wait_sem(sge[b], ge[b])

                @pl.when(jnp.logical_and(j % IB == 3, j < (NIB - 1) * IB))
                def _refill():
                    nblk = j // IB + 1
                    pltpu.sync_copy(src_hbm.at[wid * NIB + nblk],
                                    src_v.at[nblk % 2])
                    pltpu.sync_copy(dst_hbm.at[wid * NIB + nblk],
                                    dst_v.at[nblk % 2])

                def row(r, rc):
                    for c in range(D // 16):
                        s = pl.ds(c * 16, 16)
                        ms[b][r, s] = jnp.maximum(gx[b][r, s] + ge[b][r, s], 0.0)
                    return rc

                lax.fori_loop(0, K, row, 0)
                pltpu.async_copy(ms[b], acc_sh.at[idx_row(dst_v, j)],
                                 ssc[b], add=True)

                @pl.when(j + 2 < CH_PER_TILE)
                def _issue_next():
                    jn = j + 2
                    pltpu.async_copy(xe_hbm.at[idx_row(src_v, jn)],
                                     gx[b], sgx[b])
                    pltpu.async_copy(ee_hbm.at[pl.ds((ee_base + jn) * K, K)],
                                     ge[b], sge[b])
            return carry

        lax.fori_loop(0, PAIRS, pair, 0)
        for b in range(2):
            wait_sem(ssc[b], ms[b])
        plsc.subcore_barrier()

        pltpu.sync_copy(
            acc_sh.at[pl.ds(row0, ROWS_MOST)],
            out_hbm.at[cid, pl.ds(row0, ROWS_MOST)],
        )

        @pl.when(sid == NTILES - 1)
        def _out_last():
            r1 = row0 + ROWS_MOST
            pltpu.sync_copy(
                acc_sh.at[pl.ds(r1, ROWS_LAST - ROWS_MOST)],
                out_hbm.at[cid, pl.ds(r1, ROWS_LAST - ROWS_MOST)],
            )

    return body(xe, ee, src2d, dst2d)


def kernel(x, edge_index, edge_attr, Wn, bn, We, be,
           W1a, b1a, W2a, b2a, W1b, b1b, W2b, b2b):
    src2d = edge_index[0].astype(jnp.int32).reshape(NW * NIB, IB, K)
    dst2d = edge_index[1].astype(jnp.int32).reshape(NW * NIB, IB, K)

    xe = _linear(x, Wn, bn, block_rows=2000)
    ee = _linear(edge_attr, We, be, block_rows=4000)

    parts1 = _sc_aggregate(xe, ee, src2d, dst2d)
    h = _gine_mlp(xe, parts1, W1a, b1a, W2a, b2a, final_relu=True)

    parts2 = _sc_aggregate(h, ee, src2d, dst2d)
    out = _gine_mlp(h, parts2, W1b, b1b, W2b, b2b, final_relu=False)
    return out


# speedup vs baseline: 1.0984x; 1.0236x over previous
import functools

import jax
import jax.numpy as jnp
from jax import lax
from jax.experimental import pallas as pl
from jax.experimental.pallas import tpu as pltpu
from jax.experimental.pallas import tpu_sc as plsc

N = 10000
E = 320000
D = 128
DE = 16

K = 40
NCH = E // K
NTILES = 16
NW = 2 * NTILES
CH_PER_TILE = NCH // NW
ROWS_MOST = 624
ROWS_LAST = N - 15 * ROWS_MOST
ZCH = 16
IB = 25
NIB = CH_PER_TILE // IB
PAIRS = CH_PER_TILE // 2


def _lin_body(x_ref, w_ref, b_ref, o_ref):
    o_ref[...] = (
        jnp.dot(x_ref[...], w_ref[...], preferred_element_type=jnp.float32)
        + b_ref[...]
    )


def _linear(x, w, b, block_rows):
    rows, din = x.shape
    dout = w.shape[1]
    grid = rows // block_rows
    return pl.pallas_call(
        _lin_body,
        grid=(grid,),
        in_specs=[
            pl.BlockSpec((block_rows, din), lambda i: (i, 0)),
            pl.BlockSpec((din, dout), lambda i: (0, 0)),
            pl.BlockSpec((1, dout), lambda i: (0, 0)),
        ],
        out_specs=pl.BlockSpec((block_rows, dout), lambda i: (i, 0)),
        out_shape=jax.ShapeDtypeStruct((rows, dout), jnp.float32),
    )(x, w, b.reshape(1, dout))


def _enc_body(x_ref, ea_ref, wn_ref, bn_ref, we_ref, be_ref, xe_ref, ee_ref):
    xe_ref[...] = (
        jnp.dot(x_ref[...], wn_ref[...], preferred_element_type=jnp.float32)
        + bn_ref[...]
    )
    ee_ref[...] = (
        jnp.dot(ea_ref[...], we_ref[...], preferred_element_type=jnp.float32)
        + be_ref[...]
    )


def _encoders(x, ea, wn, bn, we, be):
    grid = 50
    brn = N // grid
    bre = E // grid
    return pl.pallas_call(
        _enc_body,
        grid=(grid,),
        in_specs=[
            pl.BlockSpec((brn, D), lambda i: (i, 0)),
            pl.BlockSpec((bre, DE), lambda i: (i, 0)),
            pl.BlockSpec((D, D), lambda i: (0, 0)),
            pl.BlockSpec((1, D), lambda i: (0, 0)),
            pl.BlockSpec((DE, D), lambda i: (0, 0)),
            pl.BlockSpec((1, D), lambda i: (0, 0)),
        ],
        out_specs=[
            pl.BlockSpec((brn, D), lambda i: (i, 0)),
            pl.BlockSpec((bre, D), lambda i: (i, 0)),
        ],
        out_shape=[
            jax.ShapeDtypeStruct((N, D), jnp.float32),
            jax.ShapeDtypeStruct((E, D), jnp.float32),
        ],
    )(x, ea, wn, bn.reshape(1, D), we, be.reshape(1, D))


def _mlp_body(h_ref, a_ref, w1_ref, b1_ref, w2_ref, b2_ref, o_ref, *, final_relu):
    h = h_ref[...] + a_ref[0] + a_ref[1]
    t = jnp.maximum(
        jnp.dot(h, w1_ref[...], preferred_element_type=jnp.float32) + b1_ref[...],
        0.0,
    )
    o = jnp.dot(t, w2_ref[...], preferred_element_type=jnp.float32) + b2_ref[...]
    if final_relu:
        o = jnp.maximum(o, 0.0)
    o_ref[...] = o


def _gine_mlp(h, agg2, w1, b1, w2, b2, final_relu):
    block_rows = 2000
    grid = N // block_rows
    return pl.pallas_call(
        functools.partial(_mlp_body, final_relu=final_relu),
        grid=(grid,),
        in_specs=[
            pl.BlockSpec((block_rows, D), lambda i: (i, 0)),
            pl.BlockSpec((2, block_rows, D), lambda i: (0, i, 0)),
            pl.BlockSpec((D, D), lambda i: (0, 0)),
            pl.BlockSpec((1, D), lambda i: (0, 0)),
            pl.BlockSpec((D, D), lambda i: (0, 0)),
            pl.BlockSpec((1, D), lambda i: (0, 0)),
        ],
        out_specs=pl.BlockSpec((block_rows, D), lambda i: (i, 0)),
        out_shape=jax.ShapeDtypeStruct((N, D), jnp.float32),
    )(h, agg2, w1, b1.reshape(1, D), w2, b2.reshape(1, D))


def _sc_aggregate(xe, ee, src2d, dst2d):
    mesh = plsc.VectorSubcoreMesh(core_axis_name="c", subcore_axis_name="s")

    @functools.partial(
        pl.kernel,
        out_type=jax.ShapeDtypeStruct((2, N, D), jnp.float32),
        mesh=mesh,
        scratch_types=[
            pltpu.VMEM((2, IB, K), jnp.int32),
            pltpu.VMEM((2, IB, K), jnp.int32),
            pltpu.VMEM((K, D), jnp.float32),
            pltpu.VMEM((K, D), jnp.float32),
            pltpu.VMEM((K, D), jnp.float32),
            pltpu.VMEM((K, D), jnp.float32),
            pltpu.VMEM((K, D), jnp.float32),
            pltpu.VMEM((K, D), jnp.float32),
            pltpu.VMEM_SHARED((N, D), jnp.float32),
            pltpu.SemaphoreType.DMA,
            pltpu.SemaphoreType.DMA,
            pltpu.SemaphoreType.DMA,
            pltpu.SemaphoreType.DMA,
            pltpu.SemaphoreType.DMA,
            pltpu.SemaphoreType.DMA,
            pltpu.SemaphoreType.DMA,
        ],
    )
    def body(xe_hbm, ee_hbm, src_hbm, dst_hbm, out_hbm,
             src_v, dst_v, gx0_v, gx1_v, ge0_v, ge1_v, ms0_v, ms1_v,
             acc_sh, sgx0, sgx1, sge0, sge1, ssc0, ssc1, szr):
        gx = (gx0_v, gx1_v)
        ge = (ge0_v, ge1_v)
        ms = (ms0_v, ms1_v)
        sgx = (sgx0, sgx1)
        sge = (sge0, sge1)
        ssc = (ssc0, ssc1)
        cid = lax.axis_index("c")
        sid = lax.axis_index("s")
        wid = cid * NTILES + sid

        row0 = sid * ROWS_MOST
        n_zch = jnp.where(sid == NTILES - 1, ROWS_LAST // ZCH, ROWS_MOST // ZCH)

        def zrow(r, carry):
            for c in range(D // 16):
                ms0_v[r, pl.ds(c * 16, 16)] = jnp.zeros((16,), jnp.float32)
            return carry

        lax.fori_loop(0, ZCH, zrow, 0)

        def zcp(i, carry):
            pltpu.async_copy(
                ms0_v.at[pl.ds(0, ZCH)],
                acc_sh.at[pl.ds(row0 + i * ZCH, ZCH)],
                szr,
            )
            return carry

        lax.fori_loop(0, n_zch, zcp, 0)
        pltpu.make_async_copy(
            xe_hbm.at[pl.ds(0, ROWS_MOST)],
            acc_sh.at[pl.ds(row0, ROWS_MOST)],
            szr,
        ).wait()

        @pl.when(sid == NTILES - 1)
        def _drain_last():
            pltpu.make_async_copy(
                xe_hbm.at[pl.ds(0, ROWS_LAST - ROWS_MOST)],
                acc_sh.at[pl.ds(row0, ROWS_LAST - ROWS_MOST)],
                szr,
            ).wait()

        plsc.subcore_barrier()

        ee_base = wid * CH_PER_TILE

        def idx_row(ref, j):
            return ref.at[(j // IB) % 2, j % IB]

        def wait_sem(sem, buf):
            pltpu.make_async_copy(ee_hbm.at[pl.ds(0, K)], buf, sem).wait()

        pltpu.sync_copy(src_hbm.at[wid * NIB], src_v.at[0])
        pltpu.sync_copy(dst_hbm.at[wid * NIB], dst_v.at[0])
        for b in range(2):
            pltpu.async_copy(xe_hbm.at[src_v.at[0, b]], gx[b], sgx[b])
            pltpu.async_copy(ee_hbm.at[pl.ds((ee_base + b) * K, K)], ge[b], sge[b])

        def pair(j2, carry):
            for b in range(2):
                j = 2 * j2 + b
                @pl.when(j >= 2)
                def _wait_sc():
                    wait_sem(ssc[b], ms[b])

                wait_sem(sgx[b], gx[b])
                wait_sem(sge[b], ge[b])

                @pl.when(jnp.logical_and(j % IB == 3, j < (NIB - 1) * IB))
                def _refill():
                    nblk = j // IB + 1
                    pltpu.sync_copy(src_hbm.at[wid * NIB + nblk],
                                    src_v.at[nblk % 2])
                    pltpu.sync_copy(dst_hbm.at[wid * NIB + nblk],
                                    dst_v.at[nblk % 2])

                def row(r, rc):
                    for c in range(D // 16):
                        s = pl.ds(c * 16, 16)
                        ms[b][r, s] = jnp.maximum(gx[b][r, s] + ge[b][r, s], 0.0)
                    return rc

                lax.fori_loop(0, K, row, 0)
                pltpu.async_copy(ms[b], acc_sh.at[idx_row(dst_v, j)],
                                 ssc[b], add=True)

                @pl.when(j + 2 < CH_PER_TILE)
                def _issue_next():
                    jn = j + 2
                    pltpu.async_copy(xe_hbm.at[idx_row(src_v, jn)],
                                     gx[b], sgx[b])
                    pltpu.async_copy(ee_hbm.at[pl.ds((ee_base + jn) * K, K)],
                                     ge[b], sge[b])
            return carry

        lax.fori_loop(0, PAIRS, pair, 0)
        for b in range(2):
            wait_sem(ssc[b], ms[b])
        plsc.subcore_barrier()

        pltpu.sync_copy(
            acc_sh.at[pl.ds(row0, ROWS_MOST)],
            out_hbm.at[cid, pl.ds(row0, ROWS_MOST)],
        )

        @pl.when(sid == NTILES - 1)
        def _out_last():
            r1 = row0 + ROWS_MOST
            pltpu.sync_copy(
                acc_sh.at[pl.ds(r1, ROWS_LAST - ROWS_MOST)],
                out_hbm.at[cid, pl.ds(r1, ROWS_LAST - ROWS_MOST)],
            )

    return body(xe, ee, src2d, dst2d)


def kernel(x, edge_index, edge_attr, Wn, bn, We, be,
           W1a, b1a, W2a, b2a, W1b, b1b, W2b, b2b):
    src2d = edge_index[0].astype(jnp.int32).reshape(NW * NIB, IB, K)
    dst2d = edge_index[1].astype(jnp.int32).reshape(NW * NIB, IB, K)

    xe, ee = _encoders(x, edge_attr, Wn, bn, We, be)

    parts1 = _sc_aggregate(xe, ee, src2d, dst2d)
    h = _gine_mlp(xe, parts1, W1a, b1a, W2a, b2a, final_relu=True)

    parts2 = _sc_aggregate(h, ee, src2d, dst2d)
    out = _gine_mlp(h, parts2, W1b, b1b, W2b, b2b, final_relu=False)
    return out
